# Initial kernel scaffold; baseline (speedup 1.0000x reference)
import jax
import jax.numpy as jnp
from jax.experimental import pallas as pl


def kernel(user, movie_ids, title_tok, ovrv_tok, dir_tok, cast_tok, genre_tok, pcmp_tok, pcnt_tok, numeric, title_emb, ovrv_emb, dir_emb, cast_emb, genre_emb, pcmp_emb, pcnt_emb, W1, b1, W2, b2, Wout, bout):
    raise NotImplementedError("write your pallas kernel here")


# speedup vs baseline: 4.3367x; 4.3367x over previous
import functools

import jax
import jax.numpy as jnp
from jax import lax
from jax.experimental import pallas as pl
from jax.experimental.pallas import tpu as pltpu
from jax.experimental.pallas import tpu_sc as plsc

B = 16384
_NC, _NS = 2, 16
_NW = _NC * _NS
BPW = B // _NW

L_OVRV, L_CAST, L_GENRE, L_PCMP, L_PCNT = 50, 10, 3, 3, 2
D_TIT, D_OVRV, D_DIR, D_CAST, D_GENRE, D_PCMP, D_PCNT, D_NUM = (
    20, 20, 8, 10, 15, 10, 10, 7)
P_TIT, P_OVRV, P_DIR, P_CAST, P_GENRE, P_PCMP, P_PCNT, P_NUM = (
    24, 24, 8, 16, 16, 16, 16, 8)

_CW = 128


def _sc_body(mid_hbm, ttok, dtok, otokf, ctokf, gtokf, pctokf, pntokf,
             num_hbm, temb, oemb, demb, cemb, gemb, pcemb, pnemb,
             out_tit, out_ov, out_dir, out_ct, out_gn, out_pc, out_pn, out_num,
             mid_v, tt_v, dt_v, mlv,
             six, tk1, ii,
             acc24, acc16,
             buf24, buf16,
             rt, r8, r7,
             semt, sema, semb):
  sid = lax.axis_index("s")
  wid = sid * _NC + lax.axis_index("c")
  base = wid * BPW
  abase = sid * _CW
  iota16 = lax.iota(jnp.int32, 16)

  for r in range(BPW // _CW):
    pltpu.sync_copy(mid_hbm.at[pl.ds(base + r * _CW, _CW)], mid_v.at[r])

  for r in range(BPW // _CW):
    for k in range(_CW // 16):
      v = mid_v[r, pl.ds(k * 16, 16)]
      mid_v[r, pl.ds(k * 16, 16)] = jnp.maximum(v - 1, 0)

  for k in range(_CW // 16):
    ii[0, pl.ds(k * 16, 16)] = abase + k * 16 + iota16

  for r in range(BPW // _CW):
    s = pl.ds(r * _CW, _CW)
    pltpu.sync_copy(num_hbm.at[mid_v.at[r]], r7.at[s])
    pltpu.sync_copy(ttok.at[mid_v.at[r]], tt_v.at[r])
    pltpu.sync_copy(dtok.at[mid_v.at[r]], dt_v.at[r])
    pltpu.sync_copy(temb.at[tt_v.at[r]], rt.at[s])
    pltpu.sync_copy(demb.at[dt_v.at[r]], r8.at[s])
  pltpu.sync_copy(r7, out_num.at[pl.ds(base, BPW)])
  pltpu.sync_copy(rt, out_tit.at[pl.ds(base, BPW)])
  pltpu.sync_copy(r8, out_dir.at[pl.ds(base, BPW)])

  def pooled(tokf_hbm, emb_hbm, seq_len, acc, buf, out_hbm):
    for r in range(BPW // _CW):
      def _six_for(t, srow):
        for k in range(_CW // 16):
          six[srow, pl.ds(k * 16, 16)] = (
              mid_v[r, pl.ds(k * 16, 16)] * seq_len + t)

      _six_for(0, 0)
      pltpu.sync_copy(tokf_hbm.at[six.at[0]], tk1.at[0])
      pltpu.sync_copy(emb_hbm.at[tk1.at[0]], buf)
      pltpu.sync_copy(buf, acc.at[pl.ds(abase, _CW)])

      def _step(t, carry):
        _six_for(t, 0)
        pltpu.sync_copy(tokf_hbm.at[six.at[0]], tk1.at[0])
        pltpu.sync_copy(emb_hbm.at[tk1.at[0]], buf)
        pltpu.sync_copy(buf, acc.at[ii.at[0]], add=True)
        return carry

      lax.fori_loop(1, seq_len, _step, 0)
      pltpu.sync_copy(acc.at[pl.ds(abase, _CW)],
                      out_hbm.at[pl.ds(base + r * _CW, _CW)])

  pooled(otokf, oemb, L_OVRV, acc24, buf24, out_ov)
  pooled(ctokf, cemb, L_CAST, acc16, buf16, out_ct)
  pooled(gtokf, gemb, L_GENRE, acc16, buf16, out_gn)
  pooled(pctokf, pcemb, L_PCMP, acc16, buf16, out_pc)
  pooled(pntokf, pnemb, L_PCNT, acc16, buf16, out_pn)


@functools.cache
def _make_sc_gather():
  return functools.partial(
    pl.kernel,
    out_type=[
        jax.ShapeDtypeStruct((B, P_TIT), jnp.float32),
        jax.ShapeDtypeStruct((B, P_OVRV), jnp.float32),
        jax.ShapeDtypeStruct((B, P_DIR), jnp.float32),
        jax.ShapeDtypeStruct((B, P_CAST), jnp.float32),
        jax.ShapeDtypeStruct((B, P_GENRE), jnp.float32),
        jax.ShapeDtypeStruct((B, P_PCMP), jnp.float32),
        jax.ShapeDtypeStruct((B, P_PCNT), jnp.float32),
        jax.ShapeDtypeStruct((B, P_NUM), jnp.float32),
    ],
    mesh=plsc.VectorSubcoreMesh(core_axis_name="c", subcore_axis_name="s"),
    compiler_params=pltpu.CompilerParams(use_tc_tiling_on_sc=False),
    scratch_types=[
        pltpu.VMEM((BPW // _CW, _CW), jnp.int32),
        pltpu.VMEM((BPW // _CW, _CW), jnp.int32),
        pltpu.VMEM((BPW // _CW, _CW), jnp.int32),
        pltpu.VMEM((BPW // _CW, _CW), jnp.int32),
        pltpu.VMEM((2, _CW), jnp.int32),
        pltpu.VMEM((2, _CW), jnp.int32),
        pltpu.VMEM((1, _CW), jnp.int32),
        pltpu.VMEM_SHARED((_NS * _CW, P_OVRV), jnp.float32),
        pltpu.VMEM_SHARED((_NS * _CW, P_CAST), jnp.float32),
        pltpu.VMEM((_CW, P_OVRV), jnp.float32),
        pltpu.VMEM((_CW, P_CAST), jnp.float32),
        pltpu.VMEM((BPW, P_TIT), jnp.float32),
        pltpu.VMEM((BPW, P_DIR), jnp.float32),
        pltpu.VMEM((BPW, P_NUM), jnp.float32),
        pltpu.SemaphoreType.DMA,
        pltpu.SemaphoreType.DMA,
        pltpu.SemaphoreType.DMA,
    ],
  )(_sc_body)


BT = 512


def _mlp_body(tit, ovs, dire, cts, gns, pcs, pns, num, user,
              w1, b1, w2, b2, wout, bout, out):
  movie = jnp.concatenate(
      [
          tit[...][:, :D_TIT],
          ovs[...][:, :D_OVRV] * (1.0 / L_OVRV),
          dire[...][:, :D_DIR],
          cts[...][:, :D_CAST] * (1.0 / L_CAST),
          gns[...][:, :D_GENRE] * (1.0 / L_GENRE),
          pcs[...][:, :D_PCMP] * (1.0 / L_PCMP),
          pns[...][:, :D_PCNT] * (1.0 / L_PCNT),
          num[...][:, :D_NUM],
      ],
      axis=-1,
  )
  x = movie * user[...]
  h = jnp.maximum(jnp.dot(x, w1[...]) + b1[...], 0.0)
  h = jnp.maximum(jnp.dot(h, w2[...]) + b2[...], 0.0)
  out[...] = jnp.dot(h, wout[...]) + bout[...]


def _feat_spec(d):
  return pl.BlockSpec((BT, d), lambda i: (i, 0))


def _full_spec(shape):
  nd = len(shape)
  return pl.BlockSpec(shape, lambda i: (0,) * nd)


_mlp = pl.pallas_call(
    _mlp_body,
    grid=(B // BT,),
    in_specs=[
        _feat_spec(P_TIT), _feat_spec(P_OVRV), _feat_spec(P_DIR),
        _feat_spec(P_CAST), _feat_spec(P_GENRE), _feat_spec(P_PCMP),
        _feat_spec(P_PCNT), _feat_spec(P_NUM),
        _feat_spec(100),
        _full_spec((100, 512)), _full_spec((512,)),
        _full_spec((512, 256)), _full_spec((256,)),
        _full_spec((256, 1)), _full_spec((1,)),
    ],
    out_specs=pl.BlockSpec((BT, 1), lambda i: (i, 0)),
    out_shape=jax.ShapeDtypeStruct((B, 1), jnp.float32),
)


def kernel(user, movie_ids, title_tok, ovrv_tok, dir_tok, cast_tok, genre_tok,
           pcmp_tok, pcnt_tok, numeric, title_emb, ovrv_emb, dir_emb, cast_emb,
           genre_emb, pcmp_emb, pcnt_emb, W1, b1, W2, b2, Wout, bout):
  def _pad(a, p):
    return jnp.pad(a, ((0, 0), (0, p - a.shape[1])))

  tit, ovs, dire, cts, gns, pcs, pns, num = _make_sc_gather()(
      movie_ids, title_tok, dir_tok,
      ovrv_tok.reshape(-1), cast_tok.reshape(-1), genre_tok.reshape(-1),
      pcmp_tok.reshape(-1), pcnt_tok.reshape(-1), _pad(numeric, P_NUM),
      _pad(title_emb, P_TIT), _pad(ovrv_emb, P_OVRV), dir_emb,
      _pad(cast_emb, P_CAST), _pad(genre_emb, P_GENRE),
      _pad(pcmp_emb, P_PCMP), _pad(pcnt_emb, P_PCNT))
  out = _mlp(tit, ovs, dire, cts, gns, pcs, pns, num, user,
             W1, b1, W2, b2, Wout, bout)
  return out.reshape(B)
